# baseline (speedup 1.0000x reference)
import functools
import jax
import jax.numpy as jnp
from jax import lax
from jax.experimental import pallas as pl
from jax.experimental.pallas import tpu as pltpu
from jax.experimental.pallas import tpu_sc as plsc

N = 10000
E = 320000
EP = 327680
D = 128
NC = 2
NS = 16
NW = NC * NS
CH = 128
ROWS_PER_SUB = 624

_R = 1000


def _dense_a_body(x_ref, ea_ref, lw_ref, lb_ref, wn0_ref, m0_ref, m1_ref,
                  h_ref, y0_ref, g0_ref, g1_ref):
    h = jnp.dot(x_ref[...], lw_ref[...], preferred_element_type=jnp.float32)
    h = h + lb_ref[...]
    h_ref[...] = h
    y0_ref[...] = jnp.dot(h, wn0_ref[...], preferred_element_type=jnp.float32)
    ea = ea_ref[...]
    g0_ref[...] = jnp.dot(ea, m0_ref[...], preferred_element_type=jnp.float32)
    g1_ref[...] = jnp.dot(ea, m1_ref[...], preferred_element_type=jnp.float32)


def _dense_b_body(h_ref, p0_ref, p1_ref, wr0_ref, b0_ref, wn1_ref,
                  h1_ref, y1_ref):
    h = h_ref[...]
    agg = p0_ref[...] + p1_ref[...]
    h1 = jnp.dot(h, wr0_ref[...], preferred_element_type=jnp.float32)
    h1 = jnp.maximum(h1 + agg + b0_ref[...] + h, 0.0)
    h1_ref[...] = h1
    y1_ref[...] = jnp.dot(h1, wn1_ref[...], preferred_element_type=jnp.float32)


def _dense_c_body(h1_ref, p0_ref, p1_ref, wr1_ref, b1_ref, pw_ref, pb_ref,
                  out_ref):
    h1 = h1_ref[...]
    agg = p0_ref[...] + p1_ref[...]
    h2 = jnp.dot(h1, wr1_ref[...], preferred_element_type=jnp.float32)
    h2 = h2 + agg + b1_ref[...] + h1
    out_ref[...] = (jnp.dot(h2, pw_ref[...], preferred_element_type=jnp.float32)
                    + pb_ref[...])


_row_spec = pl.BlockSpec((_R, D), lambda i: (i, 0))
_w_spec = pl.BlockSpec((D, D), lambda i: (0, 0))
_b_spec = pl.BlockSpec((1, D), lambda i: (0, 0))
_g_spec = pl.BlockSpec((_R, 32), lambda i: (i, 0))

_dense_a = pl.pallas_call(
    _dense_a_body,
    grid=(N // _R,),
    in_specs=[_row_spec, _row_spec, _w_spec, _b_spec, _w_spec,
              pl.BlockSpec((D, 32), lambda i: (0, 0)),
              pl.BlockSpec((D, 32), lambda i: (0, 0))],
    out_specs=[_row_spec, _row_spec, _g_spec, _g_spec],
    out_shape=[jax.ShapeDtypeStruct((N, D), jnp.float32),
               jax.ShapeDtypeStruct((N, D), jnp.float32),
               jax.ShapeDtypeStruct((N, 32), jnp.float32),
               jax.ShapeDtypeStruct((N, 32), jnp.float32)],
)

_dense_b = pl.pallas_call(
    _dense_b_body,
    grid=(N // _R,),
    in_specs=[_row_spec, _row_spec, _row_spec, _w_spec, _b_spec, _w_spec],
    out_specs=[_row_spec, _row_spec],
    out_shape=[jax.ShapeDtypeStruct((N, D), jnp.float32),
               jax.ShapeDtypeStruct((N, D), jnp.float32)],
)

_dense_c = pl.pallas_call(
    _dense_c_body,
    grid=(N // _R,),
    in_specs=[_row_spec, _row_spec, _row_spec, _w_spec, _b_spec, _w_spec,
              _b_spec],
    out_specs=_row_spec,
    out_shape=jax.ShapeDtypeStruct((N, D), jnp.float32),
)


CPT = EP // (NW * CH)
HCPT = CPT // 2


@functools.partial(
    pl.kernel,
    out_type=[jax.ShapeDtypeStruct((N, D), jnp.float32),
              jax.ShapeDtypeStruct((N, D), jnp.float32)],
    mesh=plsc.VectorSubcoreMesh(core_axis_name="c", subcore_axis_name="s"),
    scratch_types=[
        pltpu.VMEM((HCPT, CH), jnp.int32),
        pltpu.VMEM((HCPT, CH), jnp.int32),
        pltpu.VMEM((HCPT, CH), jnp.float32),
        pltpu.VMEM((CH, D), jnp.float32),
        pltpu.VMEM((CH, D), jnp.float32),
        pltpu.VMEM_SHARED((N, D), jnp.float32),
        pltpu.SemaphoreType.DMA,
        pltpu.SemaphoreType.DMA,
    ],
)
def _sc_edge_agg(y_hbm, src_hbm, dst_hbm, gate_hbm, out0_hbm, out1_hbm,
                 src_v, dst_v, gate_v, rows_a, rows_b, agg_sh,
                 gsem, ssem):
    c = lax.axis_index("c")
    s = lax.axis_index("s")
    wid = c * NS + s

    def _zero_row(i, _):
        for k in range(D // 16):
            rows_a[i, pl.ds(k * 16, 16)] = jnp.zeros((16,), jnp.float32)
        return 0
    lax.fori_loop(0, CH, _zero_row, 0)
    r0z = s * ROWS_PER_SUB
    for j in range(ROWS_PER_SUB // CH):
        pltpu.sync_copy(rows_a, agg_sh.at[pl.ds(r0z + j * CH, CH)])
    rem = ROWS_PER_SUB % CH
    pltpu.sync_copy(rows_a.at[pl.ds(0, rem)],
                    agg_sh.at[pl.ds(r0z + ROWS_PER_SUB - rem, rem)])

    @pl.when(s == NS - 1)
    def _zero_tail():
        pltpu.sync_copy(rows_a.at[pl.ds(0, N - NS * ROWS_PER_SUB)],
                        agg_sh.at[pl.ds(NS * ROWS_PER_SUB,
                                        N - NS * ROWS_PER_SUB)])
    plsc.subcore_barrier()

    def _scale(buf, i):
        def _scale_group(rg, _):
            g16 = gate_v[i, pl.ds(rg * 16, 16)]
            r0 = rg * 16
            for t in range(16):
                splat = g16.at[jnp.full((16,), t, jnp.int32)].get(
                    mode="promise_in_bounds")
                for k in range(D // 16):
                    v = buf[r0 + t, pl.ds(k * 16, 16)]
                    buf[r0 + t, pl.ds(k * 16, 16)] = v * splat
            return 0
        lax.fori_loop(0, CH // 16, _scale_group, 0)

    npairs = HCPT // 2
    for half in range(CPT // HCPT):
        chunk0 = wid * CPT + half * HCPT
        pltpu.sync_copy(src_hbm.at[pl.ds(chunk0, HCPT)], src_v)
        pltpu.sync_copy(dst_hbm.at[pl.ds(chunk0, HCPT)], dst_v)
        pltpu.sync_copy(gate_hbm.at[pl.ds(chunk0, HCPT)], gate_v)

        pltpu.async_copy(y_hbm.at[src_v.at[0]], rows_a, gsem)

        def _pair(p, _):
            i = 2 * p
            pltpu.make_async_copy(y_hbm.at[src_v.at[i]], rows_a,
                                  gsem).wait()

            @pl.when(p > 0)
            def _drain_b():
                pltpu.make_async_copy(rows_b, agg_sh.at[dst_v.at[i - 1]],
                                      ssem).wait()
            pltpu.async_copy(y_hbm.at[src_v.at[i + 1]], rows_b, gsem)

            _scale(rows_a, i)
            pltpu.async_copy(rows_a, agg_sh.at[dst_v.at[i]], ssem, add=True)

            pltpu.make_async_copy(y_hbm.at[src_v.at[i + 1]], rows_b,
                                  gsem).wait()
            _scale(rows_b, i + 1)
            pltpu.async_copy(rows_b, agg_sh.at[dst_v.at[i + 1]], ssem,
                             add=True)
            pltpu.make_async_copy(rows_a, agg_sh.at[dst_v.at[i]],
                                  ssem).wait()

            @pl.when(p + 1 < npairs)
            def _next_a():
                pltpu.async_copy(y_hbm.at[src_v.at[i + 2]], rows_a, gsem)
            return 0
        lax.fori_loop(0, npairs, _pair, 0)

        pltpu.make_async_copy(rows_b, agg_sh.at[dst_v.at[HCPT - 1]],
                              ssem).wait()

    plsc.subcore_barrier()
    r0 = s * ROWS_PER_SUB
    tail = N - NS * ROWS_PER_SUB

    @pl.when(c == 0)
    def _publish0():
        pltpu.sync_copy(agg_sh.at[pl.ds(r0, ROWS_PER_SUB)],
                        out0_hbm.at[pl.ds(r0, ROWS_PER_SUB)])

        @pl.when(s == NS - 1)
        def _publish0_tail():
            pltpu.sync_copy(agg_sh.at[pl.ds(NS * ROWS_PER_SUB, tail)],
                            out0_hbm.at[pl.ds(NS * ROWS_PER_SUB, tail)])

    @pl.when(c == 1)
    def _publish1():
        pltpu.sync_copy(agg_sh.at[pl.ds(r0, ROWS_PER_SUB)],
                        out1_hbm.at[pl.ds(r0, ROWS_PER_SUB)])

        @pl.when(s == NS - 1)
        def _publish1_tail():
            pltpu.sync_copy(agg_sh.at[pl.ds(NS * ROWS_PER_SUB, tail)],
                            out1_hbm.at[pl.ds(NS * ROWS_PER_SUB, tail)])


def kernel(x, edge_index, edge_attr, lift_W, lift_b, Wr0, Wn0, wg0, b0,
           Wr1, Wn1, wg1, b1, proj_W, proj_b):
    pad = jnp.broadcast_to(jnp.arange(EP - E, dtype=jnp.int32) % N,
                           (2, EP - E))
    ei_p = jnp.concatenate([edge_index, pad], axis=1)
    src = ei_p[0].reshape(EP // CH, CH)
    dst = ei_p[1].reshape(EP // CH, CH)
    ea_view = edge_attr.reshape(E // 32, 128)
    eye32 = jnp.eye(32, dtype=jnp.float32)
    M0 = jnp.kron(eye32, wg0[:, None])
    M1 = jnp.kron(eye32, wg1[:, None])

    h, y0, g0v, g1v = _dense_a(x, ea_view, lift_W, lift_b.reshape(1, D),
                               Wn0, M0, M1)
    gpad = jnp.zeros((EP - E,), jnp.float32)
    gate0 = jnp.concatenate([g0v.reshape(E), gpad]).reshape(EP // CH, CH)
    gate1 = jnp.concatenate([g1v.reshape(E), gpad]).reshape(EP // CH, CH)

    p0a, p0b = _sc_edge_agg(y0, src, dst, gate0)
    h1, y1 = _dense_b(h, p0a, p0b, Wr0, b0.reshape(1, D), Wn1)

    p1a, p1b = _sc_edge_agg(y1, src, dst, gate1)
    out = _dense_c(h1, p1a, p1b, Wr1, b1.reshape(1, D),
                   proj_W, proj_b.reshape(1, D))
    return out


# speedup vs baseline: 1.0666x; 1.0666x over previous
import functools
import jax
import jax.numpy as jnp
from jax import lax
from jax.experimental import pallas as pl
from jax.experimental.pallas import tpu as pltpu
from jax.experimental.pallas import tpu_sc as plsc

N = 10000
E = 320000
EP = 327680
D = 128
NC = 2
NS = 16
NW = NC * NS
CH = 128
ROWS_PER_SUB = 624

_R = 1000


def _dense_a_body(x_ref, ea_ref, lw_ref, lb_ref, wn0_ref, m0_ref, m1_ref,
                  h_ref, y0_ref, g0_ref, g1_ref):
    h = jnp.dot(x_ref[...], lw_ref[...], preferred_element_type=jnp.float32)
    h = h + lb_ref[...]
    h_ref[...] = h
    y0_ref[...] = jnp.dot(h, wn0_ref[...], preferred_element_type=jnp.float32)
    ea = ea_ref[...]
    g0_ref[...] = jnp.dot(ea, m0_ref[...], preferred_element_type=jnp.float32)
    g1_ref[...] = jnp.dot(ea, m1_ref[...], preferred_element_type=jnp.float32)


def _dense_b_body(h_ref, p0_ref, p1_ref, wr0_ref, b0_ref, wn1_ref,
                  h1_ref, y1_ref):
    h = h_ref[...]
    agg = p0_ref[...] + p1_ref[...]
    h1 = jnp.dot(h, wr0_ref[...], preferred_element_type=jnp.float32)
    h1 = jnp.maximum(h1 + agg + b0_ref[...] + h, 0.0)
    h1_ref[...] = h1
    y1_ref[...] = jnp.dot(h1, wn1_ref[...], preferred_element_type=jnp.float32)


def _dense_c_body(h1_ref, p0_ref, p1_ref, wr1_ref, b1_ref, pw_ref, pb_ref,
                  out_ref):
    h1 = h1_ref[...]
    agg = p0_ref[...] + p1_ref[...]
    h2 = jnp.dot(h1, wr1_ref[...], preferred_element_type=jnp.float32)
    h2 = h2 + agg + b1_ref[...] + h1
    out_ref[...] = (jnp.dot(h2, pw_ref[...], preferred_element_type=jnp.float32)
                    + pb_ref[...])


_row_spec = pl.BlockSpec((_R, D), lambda i: (i, 0))
_w_spec = pl.BlockSpec((D, D), lambda i: (0, 0))
_b_spec = pl.BlockSpec((1, D), lambda i: (0, 0))
_g_spec = pl.BlockSpec((_R, 32), lambda i: (i, 0))

_dense_a = pl.pallas_call(
    _dense_a_body,
    grid=(N // _R,),
    in_specs=[_row_spec, _row_spec, _w_spec, _b_spec, _w_spec,
              pl.BlockSpec((D, 32), lambda i: (0, 0)),
              pl.BlockSpec((D, 32), lambda i: (0, 0))],
    out_specs=[_row_spec, _row_spec, _g_spec, _g_spec],
    out_shape=[jax.ShapeDtypeStruct((N, D), jnp.float32),
               jax.ShapeDtypeStruct((N, D), jnp.float32),
               jax.ShapeDtypeStruct((N, 32), jnp.float32),
               jax.ShapeDtypeStruct((N, 32), jnp.float32)],
)

_dense_b = pl.pallas_call(
    _dense_b_body,
    grid=(N // _R,),
    in_specs=[_row_spec, _row_spec, _row_spec, _w_spec, _b_spec, _w_spec],
    out_specs=[_row_spec, _row_spec],
    out_shape=[jax.ShapeDtypeStruct((N, D), jnp.float32),
               jax.ShapeDtypeStruct((N, D), jnp.float32)],
)

_dense_c = pl.pallas_call(
    _dense_c_body,
    grid=(N // _R,),
    in_specs=[_row_spec, _row_spec, _row_spec, _w_spec, _b_spec, _w_spec,
              _b_spec],
    out_specs=_row_spec,
    out_shape=jax.ShapeDtypeStruct((N, D), jnp.float32),
)


CPT = EP // (NW * CH)
HCPT = CPT // 2
GSPLIT = 4


@functools.partial(
    pl.kernel,
    out_type=[jax.ShapeDtypeStruct((N, D), jnp.float32),
              jax.ShapeDtypeStruct((N, D), jnp.float32)],
    mesh=plsc.VectorSubcoreMesh(core_axis_name="c", subcore_axis_name="s"),
    scratch_types=[
        pltpu.VMEM((HCPT, CH), jnp.int32),
        pltpu.VMEM((HCPT, CH), jnp.int32),
        pltpu.VMEM((HCPT, CH), jnp.float32),
        pltpu.VMEM((CH, D), jnp.float32),
        pltpu.VMEM((CH, D), jnp.float32),
        pltpu.VMEM_SHARED((N, D), jnp.float32),
        pltpu.SemaphoreType.DMA,
        pltpu.SemaphoreType.DMA,
    ],
)
def _sc_edge_agg(y_hbm, src_hbm, dst_hbm, gate_hbm, out0_hbm, out1_hbm,
                 src_v, dst_v, gate_v, rows_a, rows_b, agg_sh,
                 gsem, ssem):
    c = lax.axis_index("c")
    s = lax.axis_index("s")
    wid = c * NS + s

    def _zero_row(i, _):
        for k in range(D // 16):
            rows_a[i, pl.ds(k * 16, 16)] = jnp.zeros((16,), jnp.float32)
        return 0
    lax.fori_loop(0, CH, _zero_row, 0)
    r0z = s * ROWS_PER_SUB
    for j in range(ROWS_PER_SUB // CH):
        pltpu.sync_copy(rows_a, agg_sh.at[pl.ds(r0z + j * CH, CH)])
    rem = ROWS_PER_SUB % CH
    pltpu.sync_copy(rows_a.at[pl.ds(0, rem)],
                    agg_sh.at[pl.ds(r0z + ROWS_PER_SUB - rem, rem)])

    @pl.when(s == NS - 1)
    def _zero_tail():
        pltpu.sync_copy(rows_a.at[pl.ds(0, N - NS * ROWS_PER_SUB)],
                        agg_sh.at[pl.ds(NS * ROWS_PER_SUB,
                                        N - NS * ROWS_PER_SUB)])
    plsc.subcore_barrier()

    def _scale(buf, i):
        def _scale_group(rg, _):
            g16 = gate_v[i, pl.ds(rg * 16, 16)]
            r0 = rg * 16
            for t in range(16):
                splat = g16.at[jnp.full((16,), t, jnp.int32)].get(
                    mode="promise_in_bounds")
                for k in range(D // 16):
                    v = buf[r0 + t, pl.ds(k * 16, 16)]
                    buf[r0 + t, pl.ds(k * 16, 16)] = v * splat
            return 0
        lax.fori_loop(0, CH // 16, _scale_group, 0)

    SR = CH // GSPLIT

    def _fire(buf, i):
        for j in range(GSPLIT):
            pltpu.async_copy(y_hbm.at[src_v.at[i, pl.ds(j * SR, SR)]],
                             buf.at[pl.ds(j * SR, SR)], gsem)

    def _drain(buf, i):
        for j in range(GSPLIT):
            pltpu.make_async_copy(y_hbm.at[src_v.at[i, pl.ds(j * SR, SR)]],
                                  buf.at[pl.ds(j * SR, SR)], gsem).wait()

    npairs = HCPT // 2
    for half in range(CPT // HCPT):
        chunk0 = wid * CPT + half * HCPT
        pltpu.sync_copy(src_hbm.at[pl.ds(chunk0, HCPT)], src_v)
        pltpu.sync_copy(dst_hbm.at[pl.ds(chunk0, HCPT)], dst_v)
        pltpu.sync_copy(gate_hbm.at[pl.ds(chunk0, HCPT)], gate_v)

        _fire(rows_a, 0)

        def _pair(p, _):
            i = 2 * p
            _drain(rows_a, i)
            _fire(rows_b, i + 1)
            _scale(rows_a, i)
            pltpu.sync_copy(rows_a, agg_sh.at[dst_v.at[i]], add=True)

            _drain(rows_b, i + 1)

            @pl.when(p + 1 < npairs)
            def _next_a():
                _fire(rows_a, i + 2)
            _scale(rows_b, i + 1)
            pltpu.sync_copy(rows_b, agg_sh.at[dst_v.at[i + 1]], add=True)
            return 0
        lax.fori_loop(0, npairs, _pair, 0)

    plsc.subcore_barrier()
    r0 = s * ROWS_PER_SUB
    tail = N - NS * ROWS_PER_SUB

    @pl.when(c == 0)
    def _publish0():
        pltpu.sync_copy(agg_sh.at[pl.ds(r0, ROWS_PER_SUB)],
                        out0_hbm.at[pl.ds(r0, ROWS_PER_SUB)])

        @pl.when(s == NS - 1)
        def _publish0_tail():
            pltpu.sync_copy(agg_sh.at[pl.ds(NS * ROWS_PER_SUB, tail)],
                            out0_hbm.at[pl.ds(NS * ROWS_PER_SUB, tail)])

    @pl.when(c == 1)
    def _publish1():
        pltpu.sync_copy(agg_sh.at[pl.ds(r0, ROWS_PER_SUB)],
                        out1_hbm.at[pl.ds(r0, ROWS_PER_SUB)])

        @pl.when(s == NS - 1)
        def _publish1_tail():
            pltpu.sync_copy(agg_sh.at[pl.ds(NS * ROWS_PER_SUB, tail)],
                            out1_hbm.at[pl.ds(NS * ROWS_PER_SUB, tail)])


def kernel(x, edge_index, edge_attr, lift_W, lift_b, Wr0, Wn0, wg0, b0,
           Wr1, Wn1, wg1, b1, proj_W, proj_b):
    pad = jnp.broadcast_to(jnp.arange(EP - E, dtype=jnp.int32) % N,
                           (2, EP - E))
    ei_p = jnp.concatenate([edge_index, pad], axis=1)
    src = ei_p[0].reshape(EP // CH, CH)
    dst = ei_p[1].reshape(EP // CH, CH)
    ea_view = edge_attr.reshape(E // 32, 128)
    eye32 = jnp.eye(32, dtype=jnp.float32)
    M0 = jnp.kron(eye32, wg0[:, None])
    M1 = jnp.kron(eye32, wg1[:, None])

    h, y0, g0v, g1v = _dense_a(x, ea_view, lift_W, lift_b.reshape(1, D),
                               Wn0, M0, M1)
    gpad = jnp.zeros((EP - E,), jnp.float32)
    gate0 = jnp.concatenate([g0v.reshape(E), gpad]).reshape(EP // CH, CH)
    gate1 = jnp.concatenate([g1v.reshape(E), gpad]).reshape(EP // CH, CH)

    p0a, p0b = _sc_edge_agg(y0, src, dst, gate0)
    h1, y1 = _dense_b(h, p0a, p0b, Wr0, b0.reshape(1, D), Wn1)

    p1a, p1b = _sc_edge_agg(y1, src, dst, gate1)
    out = _dense_c(h1, p1a, p1b, Wr1, b1.reshape(1, D),
                   proj_W, proj_b.reshape(1, D))
    return out


# speedup vs baseline: 1.0725x; 1.0055x over previous
import functools
import jax
import jax.numpy as jnp
from jax import lax
from jax.experimental import pallas as pl
from jax.experimental.pallas import tpu as pltpu
from jax.experimental.pallas import tpu_sc as plsc

N = 10000
E = 320000
EP = 327680
D = 128
NC = 2
NS = 16
NW = NC * NS
CH = 128
ROWS_PER_SUB = 624

_R = 1000


def _dense_a_body(x_ref, ea_ref, lw_ref, lb_ref, wn0_ref, m0_ref, m1_ref,
                  h_ref, y0_ref, g0_ref, g1_ref):
    h = jnp.dot(x_ref[...], lw_ref[...], preferred_element_type=jnp.float32)
    h = h + lb_ref[...]
    h_ref[...] = h
    y0_ref[...] = jnp.dot(h, wn0_ref[...], preferred_element_type=jnp.float32)
    ea = ea_ref[...]
    g0_ref[...] = jnp.dot(ea, m0_ref[...], preferred_element_type=jnp.float32)
    g1_ref[...] = jnp.dot(ea, m1_ref[...], preferred_element_type=jnp.float32)


def _dense_b_body(h_ref, p0_ref, p1_ref, wr0_ref, b0_ref, wn1_ref,
                  h1_ref, y1_ref):
    h = h_ref[...]
    agg = p0_ref[...] + p1_ref[...]
    h1 = jnp.dot(h, wr0_ref[...], preferred_element_type=jnp.float32)
    h1 = jnp.maximum(h1 + agg + b0_ref[...] + h, 0.0)
    h1_ref[...] = h1
    y1_ref[...] = jnp.dot(h1, wn1_ref[...], preferred_element_type=jnp.float32)


def _dense_c_body(h1_ref, p0_ref, p1_ref, wr1_ref, b1_ref, pw_ref, pb_ref,
                  out_ref):
    h1 = h1_ref[...]
    agg = p0_ref[...] + p1_ref[...]
    h2 = jnp.dot(h1, wr1_ref[...], preferred_element_type=jnp.float32)
    h2 = h2 + agg + b1_ref[...] + h1
    out_ref[...] = (jnp.dot(h2, pw_ref[...], preferred_element_type=jnp.float32)
                    + pb_ref[...])


_row_spec = pl.BlockSpec((_R, D), lambda i: (i, 0))
_w_spec = pl.BlockSpec((D, D), lambda i: (0, 0))
_b_spec = pl.BlockSpec((1, D), lambda i: (0, 0))
_g_spec = pl.BlockSpec((_R, 32), lambda i: (i, 0))

_dense_a = pl.pallas_call(
    _dense_a_body,
    grid=(N // _R,),
    in_specs=[_row_spec, _row_spec, _w_spec, _b_spec, _w_spec,
              pl.BlockSpec((D, 32), lambda i: (0, 0)),
              pl.BlockSpec((D, 32), lambda i: (0, 0))],
    out_specs=[_row_spec, _row_spec, _g_spec, _g_spec],
    out_shape=[jax.ShapeDtypeStruct((N, D), jnp.float32),
               jax.ShapeDtypeStruct((N, D), jnp.float32),
               jax.ShapeDtypeStruct((N, 32), jnp.float32),
               jax.ShapeDtypeStruct((N, 32), jnp.float32)],
)

_dense_b = pl.pallas_call(
    _dense_b_body,
    grid=(N // _R,),
    in_specs=[_row_spec, _row_spec, _row_spec, _w_spec, _b_spec, _w_spec],
    out_specs=[_row_spec, _row_spec],
    out_shape=[jax.ShapeDtypeStruct((N, D), jnp.float32),
               jax.ShapeDtypeStruct((N, D), jnp.float32)],
)

_dense_c = pl.pallas_call(
    _dense_c_body,
    grid=(N // _R,),
    in_specs=[_row_spec, _row_spec, _row_spec, _w_spec, _b_spec, _w_spec,
              _b_spec],
    out_specs=_row_spec,
    out_shape=jax.ShapeDtypeStruct((N, D), jnp.float32),
)


CPT = EP // (NW * CH)
HCPT = CPT // 2
GSPLIT = 8


@functools.partial(
    pl.kernel,
    out_type=[jax.ShapeDtypeStruct((N, D), jnp.float32),
              jax.ShapeDtypeStruct((N, D), jnp.float32)],
    mesh=plsc.VectorSubcoreMesh(core_axis_name="c", subcore_axis_name="s"),
    scratch_types=[
        pltpu.VMEM((HCPT, CH), jnp.int32),
        pltpu.VMEM((HCPT, CH), jnp.int32),
        pltpu.VMEM((HCPT, CH), jnp.float32),
        pltpu.VMEM((CH, D), jnp.float32),
        pltpu.VMEM((CH, D), jnp.float32),
        pltpu.VMEM_SHARED((N, D), jnp.float32),
        pltpu.SemaphoreType.DMA,
        pltpu.SemaphoreType.DMA,
    ],
)
def _sc_edge_agg(y_hbm, src_hbm, dst_hbm, gate_hbm, out0_hbm, out1_hbm,
                 src_v, dst_v, gate_v, rows_a, rows_b, agg_sh,
                 gsem, ssem):
    c = lax.axis_index("c")
    s = lax.axis_index("s")
    wid = c * NS + s

    def _zero_row(i, _):
        for k in range(D // 16):
            rows_a[i, pl.ds(k * 16, 16)] = jnp.zeros((16,), jnp.float32)
        return 0
    lax.fori_loop(0, CH, _zero_row, 0)
    r0z = s * ROWS_PER_SUB
    for j in range(ROWS_PER_SUB // CH):
        pltpu.sync_copy(rows_a, agg_sh.at[pl.ds(r0z + j * CH, CH)])
    rem = ROWS_PER_SUB % CH
    pltpu.sync_copy(rows_a.at[pl.ds(0, rem)],
                    agg_sh.at[pl.ds(r0z + ROWS_PER_SUB - rem, rem)])

    @pl.when(s == NS - 1)
    def _zero_tail():
        pltpu.sync_copy(rows_a.at[pl.ds(0, N - NS * ROWS_PER_SUB)],
                        agg_sh.at[pl.ds(NS * ROWS_PER_SUB,
                                        N - NS * ROWS_PER_SUB)])
    plsc.subcore_barrier()

    def _scale(buf, i):
        def _scale_group(rg, _):
            g16 = gate_v[i, pl.ds(rg * 16, 16)]
            r0 = rg * 16
            for t in range(16):
                splat = g16.at[jnp.full((16,), t, jnp.int32)].get(
                    mode="promise_in_bounds")
                for k in range(D // 16):
                    v = buf[r0 + t, pl.ds(k * 16, 16)]
                    buf[r0 + t, pl.ds(k * 16, 16)] = v * splat
            return 0
        lax.fori_loop(0, CH // 16, _scale_group, 0)

    SR = CH // GSPLIT

    def _fire(buf, i):
        for j in range(GSPLIT):
            pltpu.async_copy(y_hbm.at[src_v.at[i, pl.ds(j * SR, SR)]],
                             buf.at[pl.ds(j * SR, SR)], gsem)

    def _drain(buf, i):
        for j in range(GSPLIT):
            pltpu.make_async_copy(y_hbm.at[src_v.at[i, pl.ds(j * SR, SR)]],
                                  buf.at[pl.ds(j * SR, SR)], gsem).wait()

    npairs = HCPT // 2
    for half in range(CPT // HCPT):
        chunk0 = wid * CPT + half * HCPT
        pltpu.sync_copy(src_hbm.at[pl.ds(chunk0, HCPT)], src_v)
        pltpu.sync_copy(dst_hbm.at[pl.ds(chunk0, HCPT)], dst_v)
        pltpu.sync_copy(gate_hbm.at[pl.ds(chunk0, HCPT)], gate_v)

        _fire(rows_a, 0)

        def _pair(p, _):
            i = 2 * p
            _drain(rows_a, i)
            _fire(rows_b, i + 1)
            _scale(rows_a, i)
            pltpu.sync_copy(rows_a, agg_sh.at[dst_v.at[i]], add=True)

            _drain(rows_b, i + 1)

            @pl.when(p + 1 < npairs)
            def _next_a():
                _fire(rows_a, i + 2)
            _scale(rows_b, i + 1)
            pltpu.sync_copy(rows_b, agg_sh.at[dst_v.at[i + 1]], add=True)
            return 0
        lax.fori_loop(0, npairs, _pair, 0)

    plsc.subcore_barrier()
    r0 = s * ROWS_PER_SUB
    tail = N - NS * ROWS_PER_SUB

    @pl.when(c == 0)
    def _publish0():
        pltpu.sync_copy(agg_sh.at[pl.ds(r0, ROWS_PER_SUB)],
                        out0_hbm.at[pl.ds(r0, ROWS_PER_SUB)])

        @pl.when(s == NS - 1)
        def _publish0_tail():
            pltpu.sync_copy(agg_sh.at[pl.ds(NS * ROWS_PER_SUB, tail)],
                            out0_hbm.at[pl.ds(NS * ROWS_PER_SUB, tail)])

    @pl.when(c == 1)
    def _publish1():
        pltpu.sync_copy(agg_sh.at[pl.ds(r0, ROWS_PER_SUB)],
                        out1_hbm.at[pl.ds(r0, ROWS_PER_SUB)])

        @pl.when(s == NS - 1)
        def _publish1_tail():
            pltpu.sync_copy(agg_sh.at[pl.ds(NS * ROWS_PER_SUB, tail)],
                            out1_hbm.at[pl.ds(NS * ROWS_PER_SUB, tail)])


def kernel(x, edge_index, edge_attr, lift_W, lift_b, Wr0, Wn0, wg0, b0,
           Wr1, Wn1, wg1, b1, proj_W, proj_b):
    pad = jnp.broadcast_to(jnp.arange(EP - E, dtype=jnp.int32) % N,
                           (2, EP - E))
    ei_p = jnp.concatenate([edge_index, pad], axis=1)
    src = ei_p[0].reshape(EP // CH, CH)
    dst = ei_p[1].reshape(EP // CH, CH)
    ea_view = edge_attr.reshape(E // 32, 128)
    eye32 = jnp.eye(32, dtype=jnp.float32)
    M0 = jnp.kron(eye32, wg0[:, None])
    M1 = jnp.kron(eye32, wg1[:, None])

    h, y0, g0v, g1v = _dense_a(x, ea_view, lift_W, lift_b.reshape(1, D),
                               Wn0, M0, M1)
    gpad = jnp.zeros((EP - E,), jnp.float32)
    gate0 = jnp.concatenate([g0v.reshape(E), gpad]).reshape(EP // CH, CH)
    gate1 = jnp.concatenate([g1v.reshape(E), gpad]).reshape(EP // CH, CH)

    p0a, p0b = _sc_edge_agg(y0, src, dst, gate0)
    h1, y1 = _dense_b(h, p0a, p0b, Wr0, b0.reshape(1, D), Wn1)

    p1a, p1b = _sc_edge_agg(y1, src, dst, gate1)
    out = _dense_c(h1, p1a, p1b, Wr1, b1.reshape(1, D),
                   proj_W, proj_b.reshape(1, D))
    return out
